# baseline (speedup 1.0000x reference)
import functools

import jax
import jax.numpy as jnp
from jax import lax
from jax.experimental import pallas as pl
from jax.experimental.pallas import tpu as pltpu
from jax.experimental.pallas import tpu_sc as plsc

N = 100000
D = 128
B = 512
STEPS = 3
EPS = 1e-10
NEG = -1e30
IMIN = -2147483647

CS = 2000
NBS = N // CS
SP = 520


def _starts_body(b_ref, out_ref, cnt_ref):
    k = pl.program_id(0)

    @pl.when(k == 0)
    def _z():
        cnt_ref[...] = jnp.zeros((B, 128), jnp.float32)

    bb = b_ref[0]
    seg = lax.broadcasted_iota(jnp.int32, (B, CS), 0)
    mt = (seg == jnp.broadcast_to(bb, (B, CS))).astype(jnp.float32)
    add = jnp.sum(mt, axis=1, keepdims=True)
    cnt_ref[...] += jnp.broadcast_to(add, (B, 128))

    @pl.when(k == NBS - 1)
    def _emit():
        row = lax.broadcasted_iota(jnp.int32, (SP, B), 0)
        col = lax.broadcasted_iota(jnp.int32, (SP, B), 1)
        lt = (col < row).astype(jnp.float32)
        starts = jax.lax.dot_general(
            lt, cnt_ref[:, 0:1], (((1,), (0,)), ((), ())),
            preferred_element_type=jnp.float32)
        out_ref[...] = starts.astype(jnp.int32)


def _starts_call(batch3d):
    return pl.pallas_call(
        _starts_body,
        grid=(NBS,),
        in_specs=[pl.BlockSpec((1, 1, CS), lambda k: (k, 0, 0))],
        out_specs=pl.BlockSpec((SP, 1), lambda k: (0, 0)),
        out_shape=jax.ShapeDtypeStruct((SP, 1), jnp.int32),
        scratch_shapes=[pltpu.VMEM((B, 128), jnp.float32)],
    )(batch3d)


def _lstm_body(h_ref, c_ref, att_ref, wih_ref, whh_ref, bih_ref, bhh_ref,
               ho_ref, co_ref):
    qst = jnp.concatenate([h_ref[...], att_ref[...]], axis=-1)
    gates = (
        jax.lax.dot_general(qst, wih_ref[...], (((1,), (1,)), ((), ())),
                            preferred_element_type=jnp.float32)
        + jax.lax.dot_general(h_ref[...], whh_ref[...], (((1,), (1,)), ((), ())),
                              preferred_element_type=jnp.float32)
        + bih_ref[...] + bhh_ref[...]
    )
    i_g = jax.nn.sigmoid(gates[:, 0:D])
    f_g = jax.nn.sigmoid(gates[:, D:2 * D])
    g_g = jnp.tanh(gates[:, 2 * D:3 * D])
    o_g = jax.nn.sigmoid(gates[:, 3 * D:4 * D])
    c_new = f_g * c_ref[...] + i_g * g_g
    co_ref[...] = c_new
    ho_ref[...] = o_g * jnp.tanh(c_new)


def _lstm_call(h, c, att, W_ih, W_hh, bih, bhh):
    full = lambda k: (0, 0)
    return pl.pallas_call(
        _lstm_body,
        grid=(1,),
        in_specs=[
            pl.BlockSpec((B, D), full), pl.BlockSpec((B, D), full),
            pl.BlockSpec((B, D), full),
            pl.BlockSpec((4 * D, 2 * D), full), pl.BlockSpec((4 * D, D), full),
            pl.BlockSpec((1, 4 * D), full), pl.BlockSpec((1, 4 * D), full),
        ],
        out_specs=[pl.BlockSpec((B, D), full), pl.BlockSpec((B, D), full)],
        out_shape=[jax.ShapeDtypeStruct((B, D), jnp.float32),
                   jax.ShapeDtypeStruct((B, D), jnp.float32)],
    )(h, c, att, W_ih, W_hh, bih, bhh)


CB = 64
SEGS_W = 16


def _dot16(xr, q8):
    t0 = xr[0] * q8[0] + xr[1] * q8[1]
    t1 = xr[2] * q8[2] + xr[3] * q8[3]
    t2 = xr[4] * q8[4] + xr[5] * q8[5]
    t3 = xr[6] * q8[6] + xr[7] * q8[7]
    return jnp.sum((t0 + t1) + (t2 + t3))


def _sc_att_body(x_hbm, starts_hbm, q_hbm, att_hbm, sbuf, qv, attbuf, xbuf):
    wid = lax.axis_index("s") * 2 + lax.axis_index("c")
    seg0 = pl.multiple_of(wid * SEGS_W, SEGS_W)
    pltpu.sync_copy(starts_hbm.at[pl.ds(seg0, 24)], sbuf)
    pltpu.sync_copy(q_hbm.at[pl.ds(seg0, SEGS_W)], qv)
    sv0 = sbuf[pl.ds(0, 16)].astype(jnp.float32)
    sv1 = sbuf[pl.ds(8, 16)]
    lane = lax.iota(jnp.int32, 16)

    def g_body(g, _):
        lo = jnp.max(jnp.where(lane == g, sv0, -1.0)).astype(jnp.int32)
        hi_a = jnp.max(jnp.where(lane == g + 1, sv0, -1.0)).astype(jnp.int32)
        hi = jnp.where(g == SEGS_W - 1, sv1[8], hi_a)
        q8 = [qv[g, pl.ds(16 * t, 16)] for t in range(8)]
        astart = (lo // 8) * 8
        nblk = jnp.where(hi > lo, (hi - astart + (CB - 1)) // CB, 0)

        def blk_body(bi, carry):
            m = carry[0]
            svec = carry[1]
            acc = list(carry[2:])
            pstart = pl.multiple_of(astart + bi * CB, 8)
            pltpu.sync_copy(x_hbm.at[pl.ds(pstart, CB)], xbuf)
            pvecs = []
            bm = jnp.float32(NEG)
            for grp in range(CB // 16):
                pvec = jnp.zeros((16,), jnp.float32)
                for j in range(16):
                    xr = [xbuf[grp * 16 + j, pl.ds(16 * t, 16)]
                          for t in range(8)]
                    p = _dot16(xr, q8)
                    pvec = jnp.where(lane == j, p, pvec)
                rows = lane + (pstart + grp * 16)
                valid = (rows >= lo) & (rows < hi)
                pvecs.append(pvec)
                bm = jnp.maximum(
                    bm, jnp.max(jnp.where(valid, pvec, NEG)))
            new_m = jnp.maximum(m, bm)
            csv = jnp.exp(jnp.broadcast_to(m - new_m, (16,)))
            svec = svec * csv
            acc = [a * csv for a in acc]
            for grp in range(CB // 16):
                rows = lane + (pstart + grp * 16)
                valid = (rows >= lo) & (rows < hi)
                ev = jnp.where(valid, jnp.exp(pvecs[grp] - new_m), 0.0)
                svec = svec + ev
                for j in range(16):
                    e = ev[j]
                    for t in range(8):
                        acc[t] = acc[t] + e * xbuf[grp * 16 + j,
                                                   pl.ds(16 * t, 16)]
            return (new_m, svec, *acc)

        init = (jnp.float32(NEG), jnp.zeros((16,), jnp.float32),
                *[jnp.zeros((16,), jnp.float32) for _ in range(8)])
        res = lax.fori_loop(0, nblk, blk_body, init)
        inv = 1.0 / jnp.broadcast_to(jnp.sum(res[1]) + EPS, (16,))
        for t in range(8):
            attbuf[g, pl.ds(16 * t, 16)] = res[2 + t] * inv
        return 0

    lax.fori_loop(0, SEGS_W, g_body, 0)
    pltpu.sync_copy(attbuf, att_hbm.at[pl.ds(seg0, SEGS_W)])


@functools.partial(
    pl.kernel,
    out_type=jax.ShapeDtypeStruct((B, D), jnp.float32),
    mesh=plsc.VectorSubcoreMesh(core_axis_name="c", subcore_axis_name="s"),
    compiler_params=pltpu.CompilerParams(needs_layout_passes=False),
    scratch_types=[
        pltpu.VMEM((24,), jnp.int32),
        pltpu.VMEM((SEGS_W, D), jnp.float32),
        pltpu.VMEM((SEGS_W, D), jnp.float32),
        pltpu.VMEM((CB, D), jnp.float32),
    ],
)
def _sc_att(x_hbm, starts_hbm, q_hbm, att_hbm, sbuf, qv, attbuf, xbuf):
    _sc_att_body(x_hbm, starts_hbm, q_hbm, att_hbm, sbuf, qv, attbuf, xbuf)


@jax.jit
def kernel(x, batch, W_ih, W_hh, b_ih, b_hh):
    batch32 = batch.astype(jnp.int32)
    batch3d = batch32.reshape(NBS, 1, CS)
    bih = b_ih.reshape(1, 4 * D).astype(jnp.float32)
    bhh = b_hh.reshape(1, 4 * D).astype(jnp.float32)
    xpad = jnp.concatenate([x, jnp.zeros((CB, D), jnp.float32)], axis=0)

    starts = _starts_call(batch3d).reshape(SP)
    h = jnp.zeros((B, D), jnp.float32)
    c = jnp.zeros((B, D), jnp.float32)
    att = jnp.zeros((B, D), jnp.float32)
    for _ in range(STEPS):
        h, c = _lstm_call(h, c, att, W_ih, W_hh, bih, bhh)
        att = _sc_att(xpad, starts, h)
    return jnp.concatenate([h, att], axis=-1)


# speedup vs baseline: 1.3161x; 1.3161x over previous
import functools

import jax
import jax.numpy as jnp
from jax import lax
from jax.experimental import pallas as pl
from jax.experimental.pallas import tpu as pltpu
from jax.experimental.pallas import tpu_sc as plsc

N = 100000
D = 128
B = 512
STEPS = 3
EPS = 1e-10
NEG = -1e30
IMIN = -2147483647

CS = 2000
NBS = N // CS
SP = 520


def _starts_body(b_ref, out_ref, cnt_ref):
    k = pl.program_id(0)

    @pl.when(k == 0)
    def _z():
        cnt_ref[...] = jnp.zeros((B, 128), jnp.float32)

    bb = b_ref[0]
    seg = lax.broadcasted_iota(jnp.int32, (B, CS), 0)
    mt = (seg == jnp.broadcast_to(bb, (B, CS))).astype(jnp.float32)
    add = jnp.sum(mt, axis=1, keepdims=True)
    cnt_ref[...] += jnp.broadcast_to(add, (B, 128))

    @pl.when(k == NBS - 1)
    def _emit():
        row = lax.broadcasted_iota(jnp.int32, (SP, B), 0)
        col = lax.broadcasted_iota(jnp.int32, (SP, B), 1)
        lt = (col < row).astype(jnp.float32)
        starts = jax.lax.dot_general(
            lt, cnt_ref[:, 0:1], (((1,), (0,)), ((), ())),
            preferred_element_type=jnp.float32)
        out_ref[...] = starts.astype(jnp.int32)


def _starts_call(batch3d):
    return pl.pallas_call(
        _starts_body,
        grid=(NBS,),
        in_specs=[pl.BlockSpec((1, 1, CS), lambda k: (k, 0, 0))],
        out_specs=pl.BlockSpec((SP, 1), lambda k: (0, 0)),
        out_shape=jax.ShapeDtypeStruct((SP, 1), jnp.int32),
        scratch_shapes=[pltpu.VMEM((B, 128), jnp.float32)],
    )(batch3d)


def _lstm_body(h_ref, c_ref, att_ref, wih_ref, whh_ref, bih_ref, bhh_ref,
               ho_ref, co_ref):
    qst = jnp.concatenate([h_ref[...], att_ref[...]], axis=-1)
    gates = (
        jax.lax.dot_general(qst, wih_ref[...], (((1,), (1,)), ((), ())),
                            preferred_element_type=jnp.float32)
        + jax.lax.dot_general(h_ref[...], whh_ref[...], (((1,), (1,)), ((), ())),
                              preferred_element_type=jnp.float32)
        + bih_ref[...] + bhh_ref[...]
    )
    i_g = jax.nn.sigmoid(gates[:, 0:D])
    f_g = jax.nn.sigmoid(gates[:, D:2 * D])
    g_g = jnp.tanh(gates[:, 2 * D:3 * D])
    o_g = jax.nn.sigmoid(gates[:, 3 * D:4 * D])
    c_new = f_g * c_ref[...] + i_g * g_g
    co_ref[...] = c_new
    ho_ref[...] = o_g * jnp.tanh(c_new)


def _lstm_call(h, c, att, W_ih, W_hh, bih, bhh):
    full = lambda k: (0, 0)
    return pl.pallas_call(
        _lstm_body,
        grid=(1,),
        in_specs=[
            pl.BlockSpec((B, D), full), pl.BlockSpec((B, D), full),
            pl.BlockSpec((B, D), full),
            pl.BlockSpec((4 * D, 2 * D), full), pl.BlockSpec((4 * D, D), full),
            pl.BlockSpec((1, 4 * D), full), pl.BlockSpec((1, 4 * D), full),
        ],
        out_specs=[pl.BlockSpec((B, D), full), pl.BlockSpec((B, D), full)],
        out_shape=[jax.ShapeDtypeStruct((B, D), jnp.float32),
                   jax.ShapeDtypeStruct((B, D), jnp.float32)],
    )(h, c, att, W_ih, W_hh, bih, bhh)


CB = 64
SEGS_W = 16


def _dot16(xr, q8):
    t0 = xr[0] * q8[0] + xr[1] * q8[1]
    t1 = xr[2] * q8[2] + xr[3] * q8[3]
    t2 = xr[4] * q8[4] + xr[5] * q8[5]
    t3 = xr[6] * q8[6] + xr[7] * q8[7]
    return jnp.sum((t0 + t1) + (t2 + t3))


def _proc_block(xref, pstart, lo_v, hi_v, q8, st):
    m = [st[0], st[1]]
    sv = [st[2], st[3]]
    acc = [list(st[4:12]), list(st[12:20])]
    ps_v = jnp.broadcast_to(pstart, (16,))
    for j in range(CB):
        par = j & 1
        xr = [xref[j, pl.ds(16 * t, 16)] for t in range(8)]
        p_v = jnp.broadcast_to(_dot16(xr, q8), (16,))
        rows = ps_v + j
        valid = (rows >= lo_v) & (rows < hi_v)
        pm = jnp.where(valid, p_v, NEG)
        new_m = jnp.maximum(m[par], pm)
        cs = jnp.exp(m[par] - new_m)
        e = jnp.where(valid, jnp.exp(pm - new_m), 0.0)
        sv[par] = sv[par] * cs + e
        acc[par] = [a * cs + e * x for a, x in zip(acc[par], xr)]
        m[par] = new_m
    return (m[0], m[1], sv[0], sv[1], *acc[0], *acc[1])


def _sc_att_body(x_hbm, starts_hbm, q_hbm, att_hbm, sbuf, qv, attbuf,
                 xb0, xb1, sem0, sem1):
    wid = lax.axis_index("s") * 2 + lax.axis_index("c")
    seg0 = pl.multiple_of(wid * SEGS_W, SEGS_W)
    pltpu.sync_copy(starts_hbm.at[pl.ds(seg0, 24)], sbuf)
    pltpu.sync_copy(q_hbm.at[pl.ds(seg0, SEGS_W)], qv)
    sv0 = sbuf[pl.ds(0, 16)].astype(jnp.float32)
    sv1 = sbuf[pl.ds(8, 16)]
    lane = lax.iota(jnp.int32, 16)

    def start_dma(b, astart, buf, sem):
        ps = pl.multiple_of(astart + b * CB, 8)
        pltpu.make_async_copy(x_hbm.at[pl.ds(ps, CB)], buf, sem).start()

    def wait_dma(buf, sem):
        pltpu.make_async_copy(x_hbm.at[pl.ds(0, CB)], buf, sem).wait()

    def g_body(g, _):
        lo = jnp.max(jnp.where(lane == g, sv0, -1.0)).astype(jnp.int32)
        hi_a = jnp.max(jnp.where(lane == g + 1, sv0, -1.0)).astype(jnp.int32)
        hi = jnp.where(g == SEGS_W - 1, sv1[8], hi_a)
        q8 = [qv[g, pl.ds(16 * t, 16)] for t in range(8)]
        astart = (lo // 8) * 8
        nblk = jnp.where(hi > lo, (hi - astart + (CB - 1)) // CB, 0)
        lo_v = jnp.broadcast_to(lo, (16,))
        hi_v = jnp.broadcast_to(hi, (16,))

        @pl.when(nblk > 0)
        def _p0():
            start_dma(0, astart, xb0, sem0)

        @pl.when(nblk > 1)
        def _p1():
            start_dma(1, astart, xb1, sem1)

        def pair_body(pi, st):
            b0 = 2 * pi
            b1 = 2 * pi + 1
            wait_dma(xb0, sem0)
            st = _proc_block(xb0, pl.multiple_of(astart + b0 * CB, 8),
                             lo_v, hi_v, q8, st)

            @pl.when(b0 + 2 < nblk)
            def _n0():
                start_dma(b0 + 2, astart, xb0, sem0)

            @pl.when(b1 < nblk)
            def _w1():
                wait_dma(xb1, sem1)

            st = _proc_block(xb1, pl.multiple_of(astart + b1 * CB, 8),
                             lo_v, hi_v, q8, st)

            @pl.when(b1 + 2 < nblk)
            def _n1():
                start_dma(b1 + 2, astart, xb1, sem1)

            return st

        zeros = [jnp.zeros((16,), jnp.float32) for _ in range(16)]
        init = (jnp.full((16,), NEG, jnp.float32),
                jnp.full((16,), NEG, jnp.float32),
                jnp.zeros((16,), jnp.float32), jnp.zeros((16,), jnp.float32),
                *zeros)
        npairs = (nblk + 1) // 2
        res = lax.fori_loop(0, npairs, pair_body, init)
        mm = jnp.maximum(res[0], res[1])
        ce = jnp.exp(res[0] - mm)
        co = jnp.exp(res[1] - mm)
        s_tot = res[2] * ce + res[3] * co
        inv = 1.0 / (s_tot + EPS)
        for t in range(8):
            attbuf[g, pl.ds(16 * t, 16)] = (res[4 + t] * ce
                                            + res[12 + t] * co) * inv
        return 0

    lax.fori_loop(0, SEGS_W, g_body, 0)
    pltpu.sync_copy(attbuf, att_hbm.at[pl.ds(seg0, SEGS_W)])


@functools.partial(
    pl.kernel,
    out_type=jax.ShapeDtypeStruct((B, D), jnp.float32),
    mesh=plsc.VectorSubcoreMesh(core_axis_name="c", subcore_axis_name="s"),
    compiler_params=pltpu.CompilerParams(needs_layout_passes=False),
    scratch_types=[
        pltpu.VMEM((24,), jnp.int32),
        pltpu.VMEM((SEGS_W, D), jnp.float32),
        pltpu.VMEM((SEGS_W, D), jnp.float32),
        pltpu.VMEM((CB, D), jnp.float32),
        pltpu.VMEM((CB, D), jnp.float32),
        pltpu.SemaphoreType.DMA,
        pltpu.SemaphoreType.DMA,
    ],
)
def _sc_att(x_hbm, starts_hbm, q_hbm, att_hbm, sbuf, qv, attbuf,
            xb0, xb1, sem0, sem1):
    _sc_att_body(x_hbm, starts_hbm, q_hbm, att_hbm, sbuf, qv, attbuf,
                 xb0, xb1, sem0, sem1)


@jax.jit
def kernel(x, batch, W_ih, W_hh, b_ih, b_hh):
    batch32 = batch.astype(jnp.int32)
    batch3d = batch32.reshape(NBS, 1, CS)
    bih = b_ih.reshape(1, 4 * D).astype(jnp.float32)
    bhh = b_hh.reshape(1, 4 * D).astype(jnp.float32)
    xpad = jnp.concatenate([x, jnp.zeros((CB, D), jnp.float32)], axis=0)

    starts = _starts_call(batch3d).reshape(SP)
    h = jnp.zeros((B, D), jnp.float32)
    c = jnp.zeros((B, D), jnp.float32)
    att = jnp.zeros((B, D), jnp.float32)
    for _ in range(STEPS):
        h, c = _lstm_call(h, c, att, W_ih, W_hh, bih, bhh)
        att = _sc_att(xpad, starts, h)
    return jnp.concatenate([h, att], axis=-1)
